# baseline (speedup 1.0000x reference)
import jax
import jax.numpy as jnp
from jax import lax
from jax.experimental import pallas as pl
from jax.experimental.pallas import tpu as pltpu
from jax.experimental.pallas import tpu_sc as plsc

B = 4096
V = 1000
CB = 128
RB = 40
NCHUNK = V // RB
NR = 4
LOOK = 2
VH = 504


def _body(idx_hbm, w_hbm, avg_hbm, wt_hbm, w_tab, sem_i, sem_o):
    cid = lax.axis_index("c")
    sid = lax.axis_index("s")
    wid = sid * 2 + cid
    c0 = wid * CB
    lanes = lax.iota(jnp.int32, 16)
    zeros16 = jnp.zeros((16,), jnp.float32)

    pltpu.sync_copy(w_hbm, w_tab)

    def fetch(ring, chunk, buf):
        pltpu.async_copy(
            idx_hbm.at[pl.ds(chunk * RB, RB), pl.ds(c0, CB)],
            ring.at[buf], sem_i.at[buf])

    def wait_in(ring, buf):
        pltpu.make_async_copy(
            idx_hbm.at[pl.ds(0, RB), pl.ds(c0, CB)],
            ring.at[buf], sem_i.at[buf]).wait()

    def phases(acc, ring, ring_o):
        def put(chunk, buf):
            pltpu.async_copy(
                ring_o.at[buf],
                wt_hbm.at[pl.ds(chunk * RB, RB), pl.ds(c0, CB)],
                sem_o.at[buf])

        def wait_out(buf):
            pltpu.make_async_copy(
                ring_o.at[buf],
                wt_hbm.at[pl.ds(0, RB), pl.ds(c0, CB)],
                sem_o.at[buf]).wait()

        def zero_acc():
            @plsc.parallel_loop(0, VH, unroll=4)
            def _zero(row):
                for u in range(CB // 16):
                    acc[row, pl.ds(u * 16, 16)] = zeros16

        def step1(chunk, buf):
            @pl.when(chunk < NCHUNK)
            def _():
                wait_in(ring, buf)

                @pl.when(chunk >= NR)
                def _():
                    wait_out(buf)

                @plsc.parallel_loop(0, RB, unroll=8)
                def _rows(r):
                    for u in range(CB // 16):
                        cvec = jnp.int32(u * 16) + lanes
                        colv = ring[buf, r, pl.ds(u * 16, 16)]
                        w = plsc.load_gather(w_tab, [colv])
                        ring_o[buf, r, pl.ds(u * 16, 16)] = w
                        m = colv < VH
                        cl = jnp.where(m, colv, 0)
                        wm = jnp.where(m, w, 0.0)
                        plsc.addupdate_scatter(acc, [cl, cvec], wm)

                put(chunk, buf)

                @pl.when(chunk + LOOK < NCHUNK)
                def _():
                    fetch(ring, chunk + LOOK, (buf + LOOK) % NR)

        def step2(chunk, buf):
            @pl.when(chunk < NCHUNK)
            def _():
                wait_in(ring, buf)

                @plsc.parallel_loop(0, RB, unroll=8)
                def _rows(r):
                    for u in range(CB // 16):
                        cvec = jnp.int32(u * 16) + lanes
                        colv = ring[buf, r, pl.ds(u * 16, 16)]
                        w = plsc.load_gather(w_tab, [colv])
                        cl = colv - (V - VH)
                        m = cl >= 0
                        cl = jnp.where(m, cl, 0)
                        w = jnp.where(m, w, 0.0)
                        plsc.addupdate_scatter(acc, [cl, cvec], w)

                @pl.when(chunk + LOOK < NCHUNK)
                def _():
                    fetch(ring, chunk + LOOK, (buf + LOOK) % NR)

        for b in range(LOOK):
            fetch(ring, b, b)
        zero_acc()

        def grp1(j, carry):
            for b in range(NR):
                step1(j * NR + b, b)
            return carry
        lax.fori_loop(0, (NCHUNK + NR - 1) // NR, grp1, 0)
        pltpu.sync_copy(acc, avg_hbm.at[pl.ds(0, VH), pl.ds(c0, CB)])
        for b in range(NR):
            wait_out(b)

        for b in range(LOOK):
            fetch(ring, b, b)
        zero_acc()

        def grp2(j, carry):
            for b in range(NR):
                step2(j * NR + b, b)
            return carry
        lax.fori_loop(0, (NCHUNK + NR - 1) // NR, grp2, 0)
        pltpu.sync_copy(acc, avg_hbm.at[pl.ds(V - VH, VH), pl.ds(c0, CB)])

    pl.run_scoped(
        phases,
        pltpu.VMEM((VH, CB), jnp.float32),
        pltpu.VMEM((NR, RB, CB), jnp.int32),
        pltpu.VMEM((NR, RB, CB), jnp.float32),
    )


@jax.jit
def kernel(indices, w_es):
    run = pl.kernel(
        _body,
        out_type=(
            jax.ShapeDtypeStruct((V, B), jnp.float32),
            jax.ShapeDtypeStruct((V, B), jnp.float32),
        ),
        mesh=plsc.VectorSubcoreMesh(
            core_axis_name="c", subcore_axis_name="s",
            num_cores=2, num_subcores=16,
        ),
        scratch_types=[
            pltpu.VMEM((V,), jnp.float32),
            pltpu.SemaphoreType.DMA((NR,)),
            pltpu.SemaphoreType.DMA((NR,)),
        ],
        compiler_params=pltpu.CompilerParams(
            use_tc_tiling_on_sc=True, needs_layout_passes=False),
    )
    idx_t = jnp.transpose(indices.astype(jnp.int32))
    avg_t, wt = run(idx_t, w_es)
    return jnp.transpose(avg_t), wt


# speedup vs baseline: 1.1899x; 1.1899x over previous
import jax
import jax.numpy as jnp
from jax import lax
from jax.experimental import pallas as pl
from jax.experimental.pallas import tpu as pltpu
from jax.experimental.pallas import tpu_sc as plsc

B = 4096
V = 1000
CB = 128
RB = 40
NCHUNK = V // RB
NR = 4
LOOK = 2
VH = 504


def _body(idx_hbm, w_hbm, avg_hbm, wt_hbm, w_tab, sem_i, sem_o):
    cid = lax.axis_index("c")
    sid = lax.axis_index("s")
    wid = sid * 2 + cid
    c0 = wid * CB
    lanes = lax.iota(jnp.int32, 16)
    zeros16 = jnp.zeros((16,), jnp.float32)

    pltpu.sync_copy(w_hbm, w_tab)

    def fetch(ring, chunk, buf):
        pltpu.async_copy(
            idx_hbm.at[pl.ds(chunk * RB, RB), pl.ds(c0, CB)],
            ring.at[buf], sem_i.at[buf])

    def wait_in(ring, buf):
        pltpu.make_async_copy(
            idx_hbm.at[pl.ds(0, RB), pl.ds(c0, CB)],
            ring.at[buf], sem_i.at[buf]).wait()

    def phases(acc, ring, ring_o):
        def put(chunk, buf):
            pltpu.async_copy(
                ring_o.at[buf],
                wt_hbm.at[pl.ds(chunk * RB, RB), pl.ds(c0, CB)],
                sem_o.at[buf])

        def wait_out(buf):
            pltpu.make_async_copy(
                ring_o.at[buf],
                wt_hbm.at[pl.ds(0, RB), pl.ds(c0, CB)],
                sem_o.at[buf]).wait()

        def zero_acc():
            @plsc.parallel_loop(0, VH, unroll=4)
            def _zero(row):
                for u in range(CB // 16):
                    acc[row, pl.ds(u * 16, 16)] = zeros16

        def step1(chunk, buf):
            @pl.when(chunk < NCHUNK)
            def _():
                wait_in(ring, buf)

                @pl.when(chunk >= NR)
                def _():
                    wait_out(buf)

                @plsc.parallel_loop(0, RB, unroll=2)
                def _rows(r):
                    for u in range(CB // 16):
                        cvec = jnp.int32(u * 16) + lanes
                        colv = ring[buf, r, pl.ds(u * 16, 16)]
                        w = plsc.load_gather(w_tab, [colv])
                        ring_o[buf, r, pl.ds(u * 16, 16)] = w
                        m = colv < VH
                        cl = jnp.where(m, colv, 0)
                        wm = jnp.where(m, w, 0.0)
                        plsc.addupdate_scatter(acc, [cl, cvec], wm)

                put(chunk, buf)

                @pl.when(chunk + LOOK < NCHUNK)
                def _():
                    fetch(ring, chunk + LOOK, (buf + LOOK) % NR)

        def step2(chunk, buf):
            @pl.when(chunk < NCHUNK)
            def _():
                wait_in(ring, buf)

                @plsc.parallel_loop(0, RB, unroll=2)
                def _rows(r):
                    for u in range(CB // 16):
                        cvec = jnp.int32(u * 16) + lanes
                        colv = ring[buf, r, pl.ds(u * 16, 16)]
                        w = plsc.load_gather(w_tab, [colv])
                        cl = colv - (V - VH)
                        m = cl >= 0
                        cl = jnp.where(m, cl, 0)
                        w = jnp.where(m, w, 0.0)
                        plsc.addupdate_scatter(acc, [cl, cvec], w)

                @pl.when(chunk + LOOK < NCHUNK)
                def _():
                    fetch(ring, chunk + LOOK, (buf + LOOK) % NR)

        for b in range(LOOK):
            fetch(ring, b, b)
        zero_acc()

        def grp1(j, carry):
            for b in range(NR):
                step1(j * NR + b, b)
            return carry
        lax.fori_loop(0, (NCHUNK + NR - 1) // NR, grp1, 0)
        pltpu.sync_copy(acc, avg_hbm.at[pl.ds(0, VH), pl.ds(c0, CB)])
        for b in range(NR):
            wait_out(b)

        for b in range(LOOK):
            fetch(ring, b, b)
        zero_acc()

        def grp2(j, carry):
            for b in range(NR):
                step2(j * NR + b, b)
            return carry
        lax.fori_loop(0, (NCHUNK + NR - 1) // NR, grp2, 0)
        pltpu.sync_copy(acc, avg_hbm.at[pl.ds(V - VH, VH), pl.ds(c0, CB)])

    pl.run_scoped(
        phases,
        pltpu.VMEM((VH, CB), jnp.float32),
        pltpu.VMEM((NR, RB, CB), jnp.int32),
        pltpu.VMEM((NR, RB, CB), jnp.float32),
    )


@jax.jit
def kernel(indices, w_es):
    run = pl.kernel(
        _body,
        out_type=(
            jax.ShapeDtypeStruct((V, B), jnp.float32),
            jax.ShapeDtypeStruct((V, B), jnp.float32),
        ),
        mesh=plsc.VectorSubcoreMesh(
            core_axis_name="c", subcore_axis_name="s",
            num_cores=2, num_subcores=16,
        ),
        scratch_types=[
            pltpu.VMEM((V,), jnp.float32),
            pltpu.SemaphoreType.DMA((NR,)),
            pltpu.SemaphoreType.DMA((NR,)),
        ],
        compiler_params=pltpu.CompilerParams(
            use_tc_tiling_on_sc=True, needs_layout_passes=False),
    )
    idx_t = jnp.transpose(indices.astype(jnp.int32))
    avg_t, wt = run(idx_t, w_es)
    return jnp.transpose(avg_t), wt
